# Initial kernel scaffold; baseline (speedup 1.0000x reference)
import jax
import jax.numpy as jnp
from jax.experimental import pallas as pl


def kernel(inputs, token_table, position_embedding):
    raise NotImplementedError("write your pallas kernel here")


# speedup vs baseline: 4.1768x; 4.1768x over previous
import functools

import jax
import jax.numpy as jnp
from jax import lax
from jax.experimental import pallas as pl
from jax.experimental.pallas import tpu as pltpu
from jax.experimental.pallas import tpu_sc as plsc

B = 1024
S = 512
EMB = 128
HALF = 256
LANES = 16
NW = 32
CHUNKS_PER_W = (B * S) // (HALF * NW)


def _emb_kernel(table_hbm, idx_hbm, pos_hbm, out_hbm,
                pos_v, idx_v, rows_v, sem):
    cid = lax.axis_index("c")
    sid = lax.axis_index("s")
    wid = sid * 2 + cid
    half = wid % 2
    bgrp = wid // 2

    pltpu.sync_copy(pos_hbm.at[pl.ds(half * HALF, HALF)], pos_v)

    def chunk_body(i, carry):
        tok0 = (bgrp * 64 + i) * S + half * HALF
        pltpu.sync_copy(idx_hbm.at[pl.ds(tok0, 128)], idx_v.at[0])
        pltpu.sync_copy(idx_hbm.at[pl.ds(tok0 + 128, 128)], idx_v.at[1])
        cp0 = pltpu.make_async_copy(
            table_hbm.at[idx_v.at[0]], rows_v.at[pl.ds(0, 128)], sem)
        cp1 = pltpu.make_async_copy(
            table_hbm.at[idx_v.at[1]], rows_v.at[pl.ds(128, 128)], sem)
        cp0.start()
        cp1.start()
        cp0.wait()
        cp1.wait()

        def add_body(t, c):
            for j in range(EMB // LANES):
                sl = pl.ds(j * LANES, LANES)
                plsc.addupdate(rows_v.at[t, sl], pos_v[t, sl])
            return c
        lax.fori_loop(0, HALF, add_body, 0)

        pltpu.sync_copy(rows_v, out_hbm.at[pl.ds(tok0, HALF)])
        return carry

    lax.fori_loop(0, CHUNKS_PER_W, chunk_body, 0)


@jax.jit
def _emb(table, idx_flat, pos):
    mesh = plsc.VectorSubcoreMesh(core_axis_name="c", subcore_axis_name="s")
    return pl.kernel(
        _emb_kernel,
        mesh=mesh,
        out_type=jax.ShapeDtypeStruct((B * S, EMB), jnp.float32),
        scratch_types=[
            pltpu.VMEM((HALF, EMB), jnp.float32),
            pltpu.VMEM((2, 128), jnp.int32),
            pltpu.VMEM((HALF, EMB), jnp.float32),
            pltpu.SemaphoreType.DMA,
        ],
    )(table, idx_flat, pos)


def kernel(inputs, token_table, position_embedding):
    idx_flat = inputs.reshape(-1).astype(jnp.int32)
    out = _emb(token_table, idx_flat, position_embedding)
    return out.reshape(B, S, EMB)


# speedup vs baseline: 8.9702x; 2.1476x over previous
import jax
import jax.numpy as jnp
from jax import lax
from jax.experimental import pallas as pl
from jax.experimental.pallas import tpu as pltpu
from jax.experimental.pallas import tpu_sc as plsc

B = 1024
S = 512
EMB = 128
CHUNK = 128
LANES = 16
NW = 32
NBUF = 4
CHUNKS_PER_W = (B * S) // (CHUNK * NW)
SEQ_PER_W = 64


def _emb_kernel(table_hbm, idxarr_hbm, pos_hbm, out_hbm,
                pos_v, idx_v, rows0, rows1, rows2, rows3,
                g0, g1, g2, g3, o0, o1, o2, o3):
    cid = lax.axis_index("c")
    sid = lax.axis_index("s")
    wid = sid * 2 + cid
    half = cid
    bgrp = sid

    rows = [rows0, rows1, rows2, rows3]
    gsem = [g0, g1, g2, g3]
    osem = [o0, o1, o2, o3]

    pltpu.sync_copy(pos_hbm.at[pl.ds(half * 256, 256)], pos_v)
    pltpu.sync_copy(idxarr_hbm.at[wid], idx_v)

    seq0 = bgrp * SEQ_PER_W

    def fire_gather(i, j, bb):
        pltpu.make_async_copy(
            table_hbm.at[idx_v.at[i, j]], rows[bb], gsem[bb]).start()

    fire_gather(0, 0, 0)
    fire_gather(0, 1, 1)

    def outer(g, carry):
        for bb in range(NBUF):
            j = bb % 2
            k = g * NBUF + bb

            nb = (bb + 2) % NBUF

            @pl.when(k + 2 < CHUNKS_PER_W)
            def _():
                @pl.when(k >= 2)
                def _():
                    pltpu.make_async_copy(
                        rows[nb], out_hbm.at[pl.ds(0, CHUNK)], osem[nb]
                    ).wait()
                fire_gather((k + 2) // 2, j, nb)

            pltpu.make_async_copy(
                table_hbm.at[idx_v.at[0, 0]], rows[bb], gsem[bb]).wait()

            poff = j * CHUNK

            def add_body(t4, c):
                for dt in range(4):
                    t = t4 * 4 + dt
                    for v in range(EMB // LANES):
                        sl = pl.ds(v * LANES, LANES)
                        plsc.addupdate(rows[bb].at[t, sl],
                                       pos_v[poff + t, sl])
                return c
            lax.fori_loop(0, CHUNK // 4, add_body, 0)

            i = g * 2 + bb // 2
            tok0 = (seq0 + i) * S + half * 256 + j * CHUNK
            pltpu.make_async_copy(
                rows[bb], out_hbm.at[pl.ds(tok0, CHUNK)], osem[bb]).start()
        return carry

    lax.fori_loop(0, CHUNKS_PER_W // NBUF, outer, 0)

    for bb in range(NBUF):
        pltpu.make_async_copy(
            rows[bb], out_hbm.at[pl.ds(0, CHUNK)], osem[bb]).wait()


@jax.jit
def _emb(table, idxarr, pos):
    mesh = plsc.VectorSubcoreMesh(core_axis_name="c", subcore_axis_name="s")
    return pl.kernel(
        _emb_kernel,
        mesh=mesh,
        out_type=jax.ShapeDtypeStruct((B * S, EMB), jnp.float32),
        scratch_types=[
            pltpu.VMEM((256, EMB), jnp.float32),
            pltpu.VMEM((SEQ_PER_W, 2, 128), jnp.int32),
            pltpu.VMEM((CHUNK, EMB), jnp.float32),
            pltpu.VMEM((CHUNK, EMB), jnp.float32),
            pltpu.VMEM((CHUNK, EMB), jnp.float32),
            pltpu.VMEM((CHUNK, EMB), jnp.float32),
            pltpu.SemaphoreType.DMA,
            pltpu.SemaphoreType.DMA,
            pltpu.SemaphoreType.DMA,
            pltpu.SemaphoreType.DMA,
            pltpu.SemaphoreType.DMA,
            pltpu.SemaphoreType.DMA,
            pltpu.SemaphoreType.DMA,
            pltpu.SemaphoreType.DMA,
        ],
    )(table, idxarr, pos)


def kernel(inputs, token_table, position_embedding):
    idxarr = (inputs.astype(jnp.int32)
              .reshape(16, 64, 2, 2, 128)
              .transpose(0, 2, 1, 3, 4)
              .reshape(NW, SEQ_PER_W, 2, 128))
    out = _emb(token_table, idxarr, position_embedding)
    return out.reshape(B, S, EMB)


# speedup vs baseline: 8.9842x; 1.0016x over previous
import jax
import jax.numpy as jnp
from jax import lax
from jax.experimental import pallas as pl
from jax.experimental.pallas import tpu as pltpu
from jax.experimental.pallas import tpu_sc as plsc

B = 1024
S = 512
EMB = 128
CHUNK = 128
LANES = 16
NW = 32
NBUF = 4
CHUNKS_PER_W = (B * S) // (CHUNK * NW)
SEQ_PER_W = 64


def _emb_kernel(table_hbm, idxarr_hbm, pos_hbm, out_hbm,
                pos_v, idx_v, rows0, rows1, rows2, rows3,
                g0, g1, g2, g3, o0, o1, o2, o3):
    cid = lax.axis_index("c")
    sid = lax.axis_index("s")
    wid = sid * 2 + cid
    half = cid
    bgrp = sid

    rows = [rows0, rows1, rows2, rows3]
    gsem = [g0, g1, g2, g3]
    osem = [o0, o1, o2, o3]

    pltpu.sync_copy(pos_hbm.at[pl.ds(half * 256, 256)], pos_v)
    pltpu.sync_copy(idxarr_hbm.at[wid], idx_v)

    seq0 = bgrp * SEQ_PER_W

    def fire_gather(i, j, bb):
        pltpu.make_async_copy(
            table_hbm.at[idx_v.at[i, j]], rows[bb], gsem[bb]).start()

    fire_gather(0, 0, 0)
    fire_gather(0, 1, 1)

    def outer(g, carry):
        for bb in range(NBUF):
            j = bb % 2
            k = g * NBUF + bb

            nb = (bb + 2) % NBUF

            @pl.when(k + 2 < CHUNKS_PER_W)
            def _():
                @pl.when(k >= 2)
                def _():
                    pltpu.make_async_copy(
                        rows[nb], out_hbm.at[pl.ds(0, CHUNK)], osem[nb]
                    ).wait()
                fire_gather((k + 2) // 2, j, nb)

            pltpu.make_async_copy(
                table_hbm.at[idx_v.at[0, 0]], rows[bb], gsem[bb]).wait()

            poff = j * CHUNK

            rbuf = rows[bb]

            @plsc.parallel_loop(0, CHUNK, step=1, unroll=4)
            def _(t):
                for v in range(EMB // LANES):
                    sl = pl.ds(v * LANES, LANES)
                    plsc.addupdate(rbuf.at[t, sl], pos_v[poff + t, sl])

            i = g * 2 + bb // 2
            tok0 = (seq0 + i) * S + half * 256 + j * CHUNK
            pltpu.make_async_copy(
                rows[bb], out_hbm.at[pl.ds(tok0, CHUNK)], osem[bb]).start()
        return carry

    lax.fori_loop(0, CHUNKS_PER_W // NBUF, outer, 0)

    for bb in range(NBUF):
        pltpu.make_async_copy(
            rows[bb], out_hbm.at[pl.ds(0, CHUNK)], osem[bb]).wait()


@jax.jit
def _emb(table, idxarr, pos):
    mesh = plsc.VectorSubcoreMesh(core_axis_name="c", subcore_axis_name="s")
    return pl.kernel(
        _emb_kernel,
        mesh=mesh,
        out_type=jax.ShapeDtypeStruct((B * S, EMB), jnp.float32),
        scratch_types=[
            pltpu.VMEM((256, EMB), jnp.float32),
            pltpu.VMEM((SEQ_PER_W, 2, 128), jnp.int32),
            pltpu.VMEM((CHUNK, EMB), jnp.float32),
            pltpu.VMEM((CHUNK, EMB), jnp.float32),
            pltpu.VMEM((CHUNK, EMB), jnp.float32),
            pltpu.VMEM((CHUNK, EMB), jnp.float32),
            pltpu.SemaphoreType.DMA,
            pltpu.SemaphoreType.DMA,
            pltpu.SemaphoreType.DMA,
            pltpu.SemaphoreType.DMA,
            pltpu.SemaphoreType.DMA,
            pltpu.SemaphoreType.DMA,
            pltpu.SemaphoreType.DMA,
            pltpu.SemaphoreType.DMA,
        ],
    )(table, idxarr, pos)


def kernel(inputs, token_table, position_embedding):
    idxarr = (inputs.astype(jnp.int32)
              .reshape(16, 64, 2, 2, 128)
              .transpose(0, 2, 1, 3, 4)
              .reshape(NW, SEQ_PER_W, 2, 128))
    out = _emb(token_table, idxarr, position_embedding)
    return out.reshape(B, S, EMB)
